# Initial kernel scaffold; baseline (speedup 1.0000x reference)
import functools

import jax
import jax.numpy as jnp
from jax import lax
from jax.experimental import pallas as pl
from jax.experimental.pallas import tpu as pltpu
from jax.experimental.pallas import tpu_sc as plsc

B, N, H = 32, 64, 32
D, F = 5, 3
E_ROWS = 1537
E_PAD = 1544
NSPA = 512
NVAR = 5 * D
SPA_BASE = NVAR * E_PAD
TBL_ROWS = SPA_BASE + NSPA
P = B * N * N
NPASS = D * F + 1
NC, NS = 2, 16
NW = NC * NS
CHUNK = 1024
NCHUNKS = P // CHUNK
NCH_PER_W = NCHUNKS // NW
JS = CHUNK // 128


def _table_body(e_ref, w_ref, o_ref):
    k = pl.program_id(0)
    s = (k // D).astype(jnp.float32) + 1.0
    scale = 1.0 / (3.0 * s)
    o_ref[0] = jnp.dot(e_ref[...], w_ref[0],
                       preferred_element_type=jnp.float32) * scale


def _idx_body(ed_ref, sp_ref, o_ref):
    ed = ed_ref[0]
    sp = sp_ref[...]
    sp_ = jnp.where(sp == 0, 1, sp)
    sp_ = jnp.where(sp_ > 1, sp_ - 1, sp_)
    s = jnp.clip(sp_, 0, D)
    base = (s - 1) * (D * E_PAD)
    edt = ed.T
    dvec = (lax.broadcasted_iota(jnp.int32, (D * F, CHUNK), 0) // F) * E_PAD
    idx_edge = edt + dvec + base
    idx_spa = sp + SPA_BASE
    o_ref[0] = jnp.concatenate([idx_edge, idx_spa], axis=0)


def _asm_body(core_ref, ab_ref, t_ref, o_ref):
    core = core_ref[0]
    ct = core.T.reshape(H, N, N)
    ab = ab_ref[0]
    t = t_ref[0]
    ii = lax.broadcasted_iota(jnp.int32, (N + 1, N + 1), 0)
    jj = lax.broadcasted_iota(jnp.int32, (N + 1, N + 1), 1)
    border = jnp.logical_or(ii == 0, jj == 0).astype(jnp.float32)
    padded = jnp.pad(ct, ((0, 0), (1, 0), (1, 0)))
    o_ref[0] = 2.0 * ab[None] + t[:, None, None] * border[None] + padded


_sc_mesh = plsc.VectorSubcoreMesh(
    core_axis_name="c", subcore_axis_name="s", num_cores=NC, num_subcores=NS)


@functools.partial(
    pl.kernel,
    out_type=jax.ShapeDtypeStruct((P, H), jnp.float32),
    mesh=_sc_mesh,
    scratch_types=[
        pltpu.VMEM((NPASS, JS, 128), jnp.int32),
        pltpu.VMEM((CHUNK, H), jnp.float32),
        pltpu.SemaphoreType.DMA,
    ],
)
def _sc_gather(tbl_hbm, idx_hbm, out_hbm, idx_v, acc_v, sem):
    wid = lax.axis_index("s") * NC + lax.axis_index("c")

    def chunk_body(c, carry):
        g = wid * NCH_PER_W + c
        pltpu.sync_copy(idx_hbm.at[g], idx_v)
        for t in range(NPASS):
            cps = [
                pltpu.async_copy(
                    tbl_hbm.at[idx_v.at[t, j]],
                    acc_v.at[pl.ds(j * 128, 128)],
                    sem,
                    add=(t > 0),
                )
                for j in range(JS)
            ]
            for cp in cps:
                cp.wait()
        pltpu.sync_copy(acc_v, out_hbm.at[pl.ds(g * CHUNK, CHUNK)])
        return carry

    lax.fori_loop(0, NCH_PER_W, chunk_body, 0)


def kernel(input_nodes, attn_bias, spatial_pos, input_edges, attn_edge_type,
           edge_encoder_weight, edge_dis_encoder_weight,
           spatial_pos_encoder_weight, graph_token_virtual_distance_weight):
    del input_nodes, attn_edge_type

    e_pad = jnp.pad(edge_encoder_weight, ((0, E_PAD - E_ROWS), (0, 0)))
    dis_w = edge_dis_encoder_weight.reshape(-1, H, H)[:D]
    scaled = pl.pallas_call(
        _table_body,
        grid=(NVAR,),
        in_specs=[
            pl.BlockSpec((E_PAD, H), lambda k: (0, 0)),
            pl.BlockSpec((1, H, H), lambda k: (k % D, 0, 0)),
        ],
        out_specs=pl.BlockSpec((1, E_PAD, H), lambda k: (k, 0, 0)),
        out_shape=jax.ShapeDtypeStruct((NVAR, E_PAD, H), jnp.float32),
    )(e_pad, dis_w)
    table = jnp.concatenate(
        [scaled.reshape(NVAR * E_PAD, H), spatial_pos_encoder_weight], axis=0)

    edges_r = input_edges.reshape(NCHUNKS, CHUNK, D * F).astype(jnp.int32)
    spat_r = spatial_pos.reshape(NCHUNKS, CHUNK).astype(jnp.int32)
    idx = pl.pallas_call(
        _idx_body,
        grid=(NCHUNKS,),
        in_specs=[
            pl.BlockSpec((1, CHUNK, D * F), lambda k: (k, 0, 0)),
            pl.BlockSpec((1, CHUNK), lambda k: (k, 0)),
        ],
        out_specs=pl.BlockSpec((1, NPASS, CHUNK), lambda k: (k, 0, 0)),
        out_shape=jax.ShapeDtypeStruct((NCHUNKS, NPASS, CHUNK), jnp.int32),
    )(edges_r, spat_r)
    idx = idx.reshape(NCHUNKS, NPASS, JS, 128)

    core = _sc_gather(table, idx)

    core_r = core.reshape(B, N * N, H)
    out = pl.pallas_call(
        _asm_body,
        grid=(B,),
        in_specs=[
            pl.BlockSpec((1, N * N, H), lambda b: (b, 0, 0)),
            pl.BlockSpec((1, N + 1, N + 1), lambda b: (b, 0, 0)),
            pl.BlockSpec((1, H), lambda b: (0, 0)),
        ],
        out_specs=pl.BlockSpec((1, H, N + 1, N + 1), lambda b: (b, 0, 0, 0)),
        out_shape=jax.ShapeDtypeStruct((B, H, N + 1, N + 1), jnp.float32),
    )(core_r, attn_bias, graph_token_virtual_distance_weight)
    return out


# speedup vs baseline: 22.8731x; 22.8731x over previous
import functools

import jax
import jax.numpy as jnp
from jax import lax
from jax.experimental import pallas as pl
from jax.experimental.pallas import tpu as pltpu
from jax.experimental.pallas import tpu_sc as plsc

B, N, H = 32, 64, 32
D, F = 5, 3
E_ROWS = 1537
E_PAD = 1544
NSPA = 512
NVAR = 5 * D
SPA_BASE = NVAR * E_PAD
TBL_ROWS = SPA_BASE + NSPA
P = B * N * N
NPASS = D * F + 1
NC, NS = 2, 16
NW = NC * NS
CHUNK = 1024
NCHUNKS = P // CHUNK
NCH_PER_W = NCHUNKS // NW
JS = CHUNK // 128


def _table_body(e_ref, w_ref, o_ref):
    k = pl.program_id(0)
    s = (k // D).astype(jnp.float32) + 1.0
    scale = 1.0 / (3.0 * s)
    o_ref[0] = jnp.dot(e_ref[...], w_ref[0],
                       preferred_element_type=jnp.float32) * scale


def _idx_body(ed_ref, sp_ref, o_ref):
    ed = ed_ref[0]
    sp = sp_ref[0]
    sp_ = jnp.where(sp == 0, 1, sp)
    sp_ = jnp.where(sp_ > 1, sp_ - 1, sp_)
    s = jnp.clip(sp_, 0, D)
    base = (s - 1) * (D * E_PAD)
    edt = ed.T
    dvec = (lax.broadcasted_iota(jnp.int32, (D * F, CHUNK), 0) // F) * E_PAD
    idx_edge = edt + dvec + base
    idx_spa = sp + SPA_BASE
    o_ref[0] = jnp.concatenate([idx_edge, idx_spa], axis=0)


def _asm_body(core_ref, ab_ref, t_ref, o_ref):
    core = core_ref[0]
    ct = core.T.reshape(H, N, N)
    ab = ab_ref[0]
    t = t_ref[0]
    ii = lax.broadcasted_iota(jnp.int32, (N + 1, N + 1), 0)
    jj = lax.broadcasted_iota(jnp.int32, (N + 1, N + 1), 1)
    border = jnp.logical_or(ii == 0, jj == 0).astype(jnp.float32)
    padded = jnp.pad(ct, ((0, 0), (1, 0), (1, 0)))
    o_ref[0] = 2.0 * ab[None] + t[:, None, None] * border[None] + padded


@functools.cache
def _get_sc_gather():
    mesh = plsc.VectorSubcoreMesh(
        core_axis_name="c", subcore_axis_name="s",
        num_cores=NC, num_subcores=NS)

    @functools.partial(
        pl.kernel,
        out_type=jax.ShapeDtypeStruct((P, H), jnp.float32),
        mesh=mesh,
        scratch_types=[
            pltpu.VMEM((NPASS, JS, 128), jnp.int32),
            pltpu.VMEM((CHUNK, H), jnp.float32),
            pltpu.SemaphoreType.DMA,
        ],
        compiler_params=pltpu.CompilerParams(use_tc_tiling_on_sc=False),
    )
    def _sc_gather(tbl_hbm, idx_hbm, out_hbm, idx_v, acc_v, sem):
        wid = lax.axis_index("s") * NC + lax.axis_index("c")

        def chunk_body(c, carry):
            g = wid * NCH_PER_W + c
            pltpu.sync_copy(idx_hbm.at[g], idx_v)
            for t in range(NPASS):
                cps = [
                    pltpu.async_copy(
                        tbl_hbm.at[idx_v.at[t, j]],
                        acc_v.at[pl.ds(j * 128, 128)],
                        sem,
                        add=(t > 0),
                    )
                    for j in range(JS)
                ]
                for cp in cps:
                    cp.wait()
            pltpu.sync_copy(acc_v, out_hbm.at[pl.ds(g * CHUNK, CHUNK)])
            return carry

        lax.fori_loop(0, NCH_PER_W, chunk_body, 0)

    return _sc_gather


def kernel(input_nodes, attn_bias, spatial_pos, input_edges, attn_edge_type,
           edge_encoder_weight, edge_dis_encoder_weight,
           spatial_pos_encoder_weight, graph_token_virtual_distance_weight):
    del input_nodes, attn_edge_type

    e_pad = jnp.pad(edge_encoder_weight, ((0, E_PAD - E_ROWS), (0, 0)))
    dis_w = edge_dis_encoder_weight.reshape(-1, H, H)[:D]
    scaled = pl.pallas_call(
        _table_body,
        grid=(NVAR,),
        in_specs=[
            pl.BlockSpec((E_PAD, H), lambda k: (0, 0)),
            pl.BlockSpec((1, H, H), lambda k: (k % D, 0, 0)),
        ],
        out_specs=pl.BlockSpec((1, E_PAD, H), lambda k: (k, 0, 0)),
        out_shape=jax.ShapeDtypeStruct((NVAR, E_PAD, H), jnp.float32),
    )(e_pad, dis_w)
    table = jnp.concatenate(
        [scaled.reshape(NVAR * E_PAD, H), spatial_pos_encoder_weight], axis=0)

    edges_r = input_edges.reshape(NCHUNKS, CHUNK, D * F).astype(jnp.int32)
    spat_r = spatial_pos.reshape(NCHUNKS, 1, CHUNK).astype(jnp.int32)
    idx = pl.pallas_call(
        _idx_body,
        grid=(NCHUNKS,),
        in_specs=[
            pl.BlockSpec((1, CHUNK, D * F), lambda k: (k, 0, 0)),
            pl.BlockSpec((1, 1, CHUNK), lambda k: (k, 0, 0)),
        ],
        out_specs=pl.BlockSpec((1, NPASS, CHUNK), lambda k: (k, 0, 0)),
        out_shape=jax.ShapeDtypeStruct((NCHUNKS, NPASS, CHUNK), jnp.int32),
    )(edges_r, spat_r)
    idx = idx.reshape(NCHUNKS, NPASS, JS, 128)

    core = _get_sc_gather()(table, idx)

    core_r = core.reshape(B, N * N, H)
    out = pl.pallas_call(
        _asm_body,
        grid=(B,),
        in_specs=[
            pl.BlockSpec((1, N * N, H), lambda b: (b, 0, 0)),
            pl.BlockSpec((1, N + 1, N + 1), lambda b: (b, 0, 0)),
            pl.BlockSpec((1, H), lambda b: (0, 0)),
        ],
        out_specs=pl.BlockSpec((1, H, N + 1, N + 1), lambda b: (b, 0, 0, 0)),
        out_shape=jax.ShapeDtypeStruct((B, H, N + 1, N + 1), jnp.float32),
    )(core_r, attn_bias, graph_token_virtual_distance_weight)
    return out


# speedup vs baseline: 24.8896x; 1.0882x over previous
import functools

import jax
import jax.numpy as jnp
from jax import lax
from jax.experimental import pallas as pl
from jax.experimental.pallas import tpu as pltpu
from jax.experimental.pallas import tpu_sc as plsc

B, N, H = 32, 64, 32
D, F = 5, 3
E_ROWS = 1537
E_PAD = 1544
NSPA = 512
NVAR = 5 * D
SPA_BASE = NVAR * E_PAD
TBL_ROWS = (NVAR + 1) * E_PAD
P = B * N * N
NPASS = D * F + 1
NC, NS = 2, 16
NW = NC * NS
CHUNK = 1024
NCHUNKS = P // CHUNK
NCH_PER_W = NCHUNKS // NW
JS = CHUNK // 128


def _table_body(e_ref, w_ref, spa_ref, o_ref):
    k = pl.program_id(0)

    @pl.when(k < NVAR)
    def _():
        s = (k // D).astype(jnp.float32) + 1.0
        scale = 1.0 / (3.0 * s)
        o_ref[0] = jnp.dot(e_ref[...], w_ref[0],
                           preferred_element_type=jnp.float32) * scale

    @pl.when(k == NVAR)
    def _():
        o_ref[0] = spa_ref[...]


def _idx_body(ed_ref, sp_ref, o_ref):
    ed = ed_ref[0]
    sp = sp_ref[0]
    sp_ = jnp.where(sp == 0, 1, sp)
    sp_ = jnp.where(sp_ > 1, sp_ - 1, sp_)
    s = jnp.clip(sp_, 0, D)
    base = (s - 1) * (D * E_PAD)
    edt = ed.T
    dvec = (lax.broadcasted_iota(jnp.int32, (D * F, CHUNK), 0) // F) * E_PAD
    idx_edge = edt + dvec + base
    idx_spa = sp + SPA_BASE
    out = jnp.concatenate([idx_edge, idx_spa], axis=0)
    o_ref[0] = out.reshape(NPASS, JS, 128)


def _asm_body(core_ref, ab_ref, t_ref, o_ref):
    core = core_ref[0]
    ct = core.T.reshape(H, N, N)
    ab = ab_ref[0]
    t = t_ref[0]
    ii = lax.broadcasted_iota(jnp.int32, (N + 1, N + 1), 0)
    jj = lax.broadcasted_iota(jnp.int32, (N + 1, N + 1), 1)
    border = jnp.logical_or(ii == 0, jj == 0).astype(jnp.float32)
    padded = jnp.pad(ct, ((0, 0), (1, 0), (1, 0)))
    o_ref[0] = 2.0 * ab[None] + t[:, None, None] * border[None] + padded


@functools.cache
def _get_sc_gather():
    mesh = plsc.VectorSubcoreMesh(
        core_axis_name="c", subcore_axis_name="s",
        num_cores=NC, num_subcores=NS)

    @functools.partial(
        pl.kernel,
        out_type=jax.ShapeDtypeStruct((P, H), jnp.float32),
        mesh=mesh,
        scratch_types=[
            pltpu.VMEM((NPASS, JS, 128), jnp.int32),
            pltpu.VMEM((CHUNK, H), jnp.float32),
            pltpu.SemaphoreType.DMA,
        ],
        compiler_params=pltpu.CompilerParams(use_tc_tiling_on_sc=False),
    )
    def _sc_gather(tbl_hbm, idx_hbm, out_hbm, idx_v, acc_v, sem):
        wid = lax.axis_index("s") * NC + lax.axis_index("c")

        def chunk_body(c, carry):
            g = wid * NCH_PER_W + c
            pltpu.sync_copy(idx_hbm.at[g], idx_v)
            cps0 = [
                pltpu.async_copy(
                    tbl_hbm.at[idx_v.at[0, j]],
                    acc_v.at[pl.ds(j * 128, 128)],
                    sem,
                )
                for j in range(JS)
            ]
            for cp in cps0:
                cp.wait()
            cps = [
                pltpu.async_copy(
                    tbl_hbm.at[idx_v.at[t, j]],
                    acc_v.at[pl.ds(j * 128, 128)],
                    sem,
                    add=True,
                )
                for t in range(1, NPASS)
                for j in range(JS)
            ]
            for cp in cps:
                cp.wait()
            pltpu.sync_copy(acc_v, out_hbm.at[pl.ds(g * CHUNK, CHUNK)])
            return carry

        lax.fori_loop(0, NCH_PER_W, chunk_body, 0)

    return _sc_gather


def kernel(input_nodes, attn_bias, spatial_pos, input_edges, attn_edge_type,
           edge_encoder_weight, edge_dis_encoder_weight,
           spatial_pos_encoder_weight, graph_token_virtual_distance_weight):
    del input_nodes, attn_edge_type

    e_pad = jnp.pad(edge_encoder_weight, ((0, E_PAD - E_ROWS), (0, 0)))
    spa_pad = jnp.pad(spatial_pos_encoder_weight, ((0, E_PAD - NSPA), (0, 0)))
    dis_w = edge_dis_encoder_weight.reshape(-1, H, H)[:D]
    scaled = pl.pallas_call(
        _table_body,
        grid=(NVAR + 1,),
        in_specs=[
            pl.BlockSpec((E_PAD, H), lambda k: (0, 0)),
            pl.BlockSpec((1, H, H), lambda k: (k % D, 0, 0)),
            pl.BlockSpec((E_PAD, H), lambda k: (0, 0)),
        ],
        out_specs=pl.BlockSpec((1, E_PAD, H), lambda k: (k, 0, 0)),
        out_shape=jax.ShapeDtypeStruct((NVAR + 1, E_PAD, H), jnp.float32),
    )(e_pad, dis_w, spa_pad)
    table = scaled.reshape(TBL_ROWS, H)

    edges_r = input_edges.reshape(NCHUNKS, CHUNK, D * F).astype(jnp.int32)
    spat_r = spatial_pos.reshape(NCHUNKS, 1, CHUNK).astype(jnp.int32)
    idx = pl.pallas_call(
        _idx_body,
        grid=(NCHUNKS,),
        in_specs=[
            pl.BlockSpec((1, CHUNK, D * F), lambda k: (k, 0, 0)),
            pl.BlockSpec((1, 1, CHUNK), lambda k: (k, 0, 0)),
        ],
        out_specs=pl.BlockSpec((1, NPASS, JS, 128), lambda k: (k, 0, 0, 0)),
        out_shape=jax.ShapeDtypeStruct((NCHUNKS, NPASS, JS, 128), jnp.int32),
    )(edges_r, spat_r)

    core = _get_sc_gather()(table, idx)

    core_r = core.reshape(B, N * N, H)
    out = pl.pallas_call(
        _asm_body,
        grid=(B,),
        in_specs=[
            pl.BlockSpec((1, N * N, H), lambda b: (b, 0, 0)),
            pl.BlockSpec((1, N + 1, N + 1), lambda b: (b, 0, 0)),
            pl.BlockSpec((1, H), lambda b: (0, 0)),
        ],
        out_specs=pl.BlockSpec((1, H, N + 1, N + 1), lambda b: (b, 0, 0, 0)),
        out_shape=jax.ShapeDtypeStruct((B, H, N + 1, N + 1), jnp.float32),
    )(core_r, attn_bias, graph_token_virtual_distance_weight)
    return out
